# Initial kernel scaffold; baseline (speedup 1.0000x reference)
import jax
import jax.numpy as jnp
from jax.experimental import pallas as pl


def kernel(node_features, edge_indices, edge_features, W_node, W_edge, a):
    raise NotImplementedError("write your pallas kernel here")


# speedup vs baseline: 16.1710x; 16.1710x over previous
import functools

import jax
import jax.numpy as jnp
from jax import lax
from jax.experimental import pallas as pl
from jax.experimental.pallas import tpu as pltpu
from jax.experimental.pallas import tpu_sc as plsc

N, E, ND, ED, H, HD = 2048, 8192, 128, 16, 4, 32
NC, NS, L = 2, 16, 16
EC = E // NS
NPS = N // NS
HPC = H // NC
CW = HPC * HD
KCH = EC // 128
GPC = 128 // L
NKC = ND // L

_f32 = jnp.float32
_i32 = jnp.int32


def _prep_body(nf_ref, wn_ref, ef_ref, we_ref, a_ref,
               h_ref, alpha_ref, ae_ref, m_ref):
    nf = nf_ref[...]
    wn = wn_ref[...]
    av = a_ref[...]
    h = jnp.dot(nf, wn, preferred_element_type=_f32)
    h_ref[...] = h

    wn3 = wn.reshape(ND, H, HD)
    a_s = av[:, 0:HD]
    a_t = av[:, HD:2 * HD]
    a_e = av[:, 2 * HD:3 * HD]
    As = (wn3 * a_s[None, :, :]).sum(axis=-1)
    At = (wn3 * a_t[None, :, :]).sum(axis=-1)
    Ast = jnp.concatenate([As, At], axis=1)
    alpha = jnp.dot(nf, Ast, preferred_element_type=_f32)
    alpha_ref[...] = alpha

    we3 = we_ref[...].reshape(ED, H, HD)
    Ae = (we3 * a_e[None, :, :]).sum(axis=-1)
    ae = jnp.dot(ef_ref[...], Ae, preferred_element_type=_f32)
    ae_ref[...] = ae

    amax = jnp.max(alpha, axis=0, keepdims=True)
    aemax = jnp.max(ae, axis=0, keepdims=True)
    M = jnp.maximum(amax[:, 0:H] + amax[:, H:2 * H] + aemax, 0.0)
    M = jnp.minimum(M, 80.0)
    m_ref[...] = jnp.concatenate([M, jnp.zeros((1, L - H), _f32)], axis=1)


_prep = pl.pallas_call(
    _prep_body,
    out_shape=[
        jax.ShapeDtypeStruct((N, ND), _f32),
        jax.ShapeDtypeStruct((N, 2 * H), _f32),
        jax.ShapeDtypeStruct((E, H), _f32),
        jax.ShapeDtypeStruct((1, L), _f32),
    ],
)


def _sc_body(src_hbm, tgt_hbm, h_hbm, alpha_hbm, ae_hbm, m_hbm,
             out_hbm,
             src_v, tgt_v, src_idx, tgt_idx, alpha_v, ae_v, m_v,
             exp_v, att_v, sums2_v, sums_loc, dloc_v, denom_v, rows_v,
             stage_sh, denom_sh, hp_sh, sem):
    c = lax.axis_index("c")
    s = lax.axis_index("s")
    base = s * EC
    zero16 = jnp.zeros((L,), _f32)
    iota = lax.iota(_i32, L)

    pltpu.sync_copy(src_hbm.at[pl.ds(base, EC)], src_v)
    pltpu.sync_copy(tgt_hbm.at[pl.ds(base, EC)], tgt_v)
    pltpu.sync_copy(alpha_hbm, alpha_v)
    pltpu.sync_copy(ae_hbm.at[pl.ds(base * H, EC * H)], ae_v)
    pltpu.sync_copy(m_hbm, m_v)

    def _zero_sums(g, _):
        for i in range(HPC):
            sums2_v[i, pl.ds(g * L, L)] = zero16
        return 0
    lax.fori_loop(0, N // L, _zero_sums, 0)

    def _zero_rows(j, _):
        for k in range(NKC):
            rows_v[j, pl.ds(k * L, L)] = zero16
        return 0
    lax.fori_loop(0, NPS, _zero_rows, 0)
    pltpu.sync_copy(rows_v.at[pl.ds(0, NPS)], hp_sh.at[pl.ds(s * NPS, NPS)])

    def _fill_idx(g, _):
        k = g // GPC
        r = g - k * GPC
        sl = pl.ds(k * 128 + r * L, L)
        dsl = pl.ds(r * L, L)
        src_idx[k, dsl] = src_v[sl]
        tgt_idx[k, dsl] = tgt_v[sl]
        return 0
    lax.fori_loop(0, EC // L, _fill_idx, 0)

    plsc.subcore_barrier()

    for i in range(HPC):
        gh = 2 * c + i
        ghv = jnp.full((L,), 0, _i32) + gh
        mg = plsc.load_gather(m_v, [ghv])
        eneg = jnp.exp(-mg)
        iv = jnp.full((L,), i, _i32)

        def _logit(g, _):
            sl = pl.ds(g * L, L)
            sv = src_v[sl]
            tv = tgt_v[sl]
            jv = iota + g * L
            asv = plsc.load_gather(alpha_v, [sv * (2 * H) + gh])
            atv = plsc.load_gather(alpha_v, [tv * (2 * H) + H + gh])
            aev = plsc.load_gather(ae_v, [jv * H + gh])
            l = asv + atv + aev
            l = jnp.where(l > 0.0, l, l * 0.2)
            ex = jnp.exp(l - mg)
            exp_v[i, sl] = ex
            plsc.addupdate_scatter(sums2_v, [iv, tv], ex - eneg)
            return 0
        lax.fori_loop(0, EC // L, _logit, 0)

    pltpu.sync_copy(sums2_v, stage_sh.at[s])
    plsc.subcore_barrier()

    for s2 in range(NS):
        for i in range(HPC):
            pltpu.sync_copy(stage_sh.at[s2, i, pl.ds(s * NPS, NPS)],
                            sums_loc.at[s2, i])
    for i in range(HPC):
        gh = 2 * c + i
        ghv = jnp.full((L,), 0, _i32) + gh
        mg = plsc.load_gather(m_v, [ghv])
        escale = jnp.exp(-mg) * float(E)

        def _denom(g, _):
            sl = pl.ds(g * L, L)
            acc = escale
            for s2 in range(NS):
                acc = acc + sums_loc[s2, i, sl]
            dloc_v[i, sl] = acc
            return 0
        lax.fori_loop(0, NPS // L, _denom, 0)
        pltpu.sync_copy(dloc_v.at[i], denom_sh.at[i, pl.ds(s * NPS, NPS)])
    plsc.subcore_barrier()

    pltpu.sync_copy(denom_sh, denom_v)
    for i in range(HPC):
        iv = jnp.full((L,), i, _i32)

        def _att(g, _):
            sl = pl.ds(g * L, L)
            tv = tgt_v[sl]
            dv = plsc.load_gather(denom_v, [iv, tv])
            att_v[i, sl] = exp_v[i, sl] / dv
            return 0
        lax.fori_loop(0, EC // L, _att, 0)

    descs = [pltpu.async_copy(h_hbm.at[src_idx.at[k]],
                              rows_v.at[pl.ds(k * 128, 128)], sem)
             for k in range(KCH)]
    for d in descs:
        d.wait()

    def _scale(g, _):
        sl = pl.ds(g * L, L)
        a0v = att_v[0, sl]
        a1v = att_v[1, sl]
        avs = []
        for k in range(NKC):
            gh = k // HPC
            avs.append(jnp.where(gh == 2 * c, a0v,
                                 jnp.where(gh == 2 * c + 1, a1v, zero16)))
        for r in range(L):
            j = g * L + r
            for k in range(NKC):
                ksl = pl.ds(k * L, L)
                rows_v[j, ksl] = rows_v[j, ksl] * avs[k][r]
        return 0
    lax.fori_loop(0, EC // L, _scale, 0)

    for k in range(KCH):
        pltpu.sync_copy(rows_v.at[pl.ds(k * 128, 128)],
                        hp_sh.at[tgt_idx.at[k]], add=True)
    plsc.subcore_barrier()

    pltpu.sync_copy(hp_sh.at[pl.ds(s * NPS, NPS)],
                    out_hbm.at[c, pl.ds(s * NPS, NPS)])


@functools.lru_cache(maxsize=None)
def _build_sc_edges():
  return pl.kernel(
    _sc_body,
    out_type=jax.ShapeDtypeStruct((NC, N, ND), _f32),
    mesh=plsc.VectorSubcoreMesh(core_axis_name="c", subcore_axis_name="s",
                                num_cores=NC, num_subcores=NS),
    compiler_params=pltpu.CompilerParams(needs_layout_passes=False),
    scratch_types=[
        pltpu.VMEM((EC,), _i32),
        pltpu.VMEM((EC,), _i32),
        pltpu.VMEM((KCH, 128), _i32),
        pltpu.VMEM((KCH, 128), _i32),
        pltpu.VMEM((N * 2 * H,), _f32),
        pltpu.VMEM((EC * H,), _f32),
        pltpu.VMEM((L,), _f32),
        pltpu.VMEM((HPC, EC), _f32),
        pltpu.VMEM((HPC, EC), _f32),
        pltpu.VMEM((HPC, N), _f32),
        pltpu.VMEM((NS, HPC, NPS), _f32),
        pltpu.VMEM((HPC, NPS), _f32),
        pltpu.VMEM((HPC, N), _f32),
        pltpu.VMEM((EC, ND), _f32),
        pltpu.VMEM_SHARED((NS, HPC, N), _f32),
        pltpu.VMEM_SHARED((HPC, N), _f32),
        pltpu.VMEM_SHARED((N, ND), _f32),
        pltpu.SemaphoreType.DMA,
    ],
  )


@jax.jit
def kernel(node_features, edge_indices, edge_features, W_node, W_edge, a):
    h, alpha, ae, m = _prep(node_features, W_node, edge_features, W_edge, a)
    src = edge_indices[0]
    tgt = edge_indices[1]
    out = _build_sc_edges()(src, tgt, h, alpha.reshape(-1), ae.reshape(-1),
                            m.reshape(-1))
    return jnp.concatenate([out[0][:, :CW], out[1][:, CW:]], axis=1)


# speedup vs baseline: 17.6052x; 1.0887x over previous
import functools

import jax
import jax.numpy as jnp
from jax import lax
from jax.experimental import pallas as pl
from jax.experimental.pallas import tpu as pltpu
from jax.experimental.pallas import tpu_sc as plsc

N, E, ND, ED, H, HD = 2048, 8192, 128, 16, 4, 32
NC, NS, L = 2, 16, 16
EC = E // NS
NPS = N // NS
HPC = H // NC
CW = HPC * HD
KCH = EC // 128
GPC = 128 // L
NKC = ND // L

_f32 = jnp.float32
_i32 = jnp.int32


def _prep_body(nf_ref, wn_ref, ef_ref, we_ref, a_ref,
               h_ref, alpha_ref, ae_ref, m_ref):
    nf = nf_ref[...]
    wn = wn_ref[...]
    av = a_ref[...]
    h = jnp.dot(nf, wn, preferred_element_type=_f32)
    h_ref[...] = h

    wn3 = wn.reshape(ND, H, HD)
    a_s = av[:, 0:HD]
    a_t = av[:, HD:2 * HD]
    a_e = av[:, 2 * HD:3 * HD]
    As = (wn3 * a_s[None, :, :]).sum(axis=-1)
    At = (wn3 * a_t[None, :, :]).sum(axis=-1)
    Ast = jnp.concatenate([As, At], axis=1)
    alpha = jnp.dot(nf, Ast, preferred_element_type=_f32)
    alpha_ref[...] = alpha

    we3 = we_ref[...].reshape(ED, H, HD)
    Ae = (we3 * a_e[None, :, :]).sum(axis=-1)
    ae = jnp.dot(ef_ref[...], Ae, preferred_element_type=_f32)
    ae_ref[...] = ae

    amax = jnp.max(alpha, axis=0, keepdims=True)
    aemax = jnp.max(ae, axis=0, keepdims=True)
    M = jnp.maximum(amax[:, 0:H] + amax[:, H:2 * H] + aemax, 0.0)
    M = jnp.minimum(M, 80.0)
    m_ref[...] = jnp.concatenate([M, jnp.zeros((1, L - H), _f32)], axis=1)


_prep = pl.pallas_call(
    _prep_body,
    out_shape=[
        jax.ShapeDtypeStruct((N, ND), _f32),
        jax.ShapeDtypeStruct((N, 2 * H), _f32),
        jax.ShapeDtypeStruct((E, H), _f32),
        jax.ShapeDtypeStruct((1, L), _f32),
    ],
)


def _sc_body(src_hbm, tgt_hbm, h_hbm, alpha_hbm, ae_hbm, m_hbm,
             out_hbm,
             src_v, tgt_v, src_idx, tgt_idx, alpha_v, ae_v, m_v,
             exp_v, att_v, sums2_v, sums_loc, dloc_v, denom_v, rows_v,
             stage_sh, denom_sh, hp_sh, sem):
    c = lax.axis_index("c")
    s = lax.axis_index("s")
    base = s * EC
    zero16 = jnp.zeros((L,), _f32)
    iota = lax.iota(_i32, L)

    pltpu.sync_copy(src_hbm.at[pl.ds(base, EC)], src_v)
    pltpu.sync_copy(tgt_hbm.at[pl.ds(base, EC)], tgt_v)
    pltpu.sync_copy(alpha_hbm, alpha_v)
    pltpu.sync_copy(ae_hbm.at[pl.ds(base * H, EC * H)], ae_v)
    pltpu.sync_copy(m_hbm, m_v)

    def _zero_sums(g, _):
        for i in range(HPC):
            sums2_v[i, pl.ds(g * L, L)] = zero16
        return 0
    lax.fori_loop(0, N // L, _zero_sums, 0)

    def _zero_rows(j, _):
        for k in range(NKC):
            rows_v[j, pl.ds(k * L, L)] = zero16
        return 0
    lax.fori_loop(0, NPS, _zero_rows, 0)
    pltpu.sync_copy(rows_v.at[pl.ds(0, NPS)], hp_sh.at[pl.ds(s * NPS, NPS)])

    def _fill_idx(g, _):
        k = g // GPC
        r = g - k * GPC
        sl = pl.ds(k * 128 + r * L, L)
        dsl = pl.ds(r * L, L)
        src_idx[k, dsl] = src_v[sl]
        tgt_idx[k, dsl] = tgt_v[sl]
        return 0
    lax.fori_loop(0, EC // L, _fill_idx, 0)

    descs = [pltpu.async_copy(h_hbm.at[src_idx.at[k]],
                              rows_v.at[pl.ds(k * 128, 128)], sem)
             for k in range(KCH)]

    plsc.subcore_barrier()

    for i in range(HPC):
        gh = 2 * c + i
        ghv = jnp.full((L,), 0, _i32) + gh
        mg = plsc.load_gather(m_v, [ghv])
        eneg = jnp.exp(-mg)
        iv = jnp.full((L,), i, _i32)

        def _logit(g, _):
            sl = pl.ds(g * L, L)
            sv = src_v[sl]
            tv = tgt_v[sl]
            jv = iota + g * L
            asv = plsc.load_gather(alpha_v, [sv * (2 * H) + gh])
            atv = plsc.load_gather(alpha_v, [tv * (2 * H) + H + gh])
            aev = plsc.load_gather(ae_v, [jv * H + gh])
            l = asv + atv + aev
            l = jnp.where(l > 0.0, l, l * 0.2)
            ex = jnp.exp(l - mg)
            exp_v[i, sl] = ex
            plsc.addupdate_scatter(sums2_v, [iv, tv], ex - eneg)
            return 0
        lax.fori_loop(0, EC // L, _logit, 0)

    pltpu.sync_copy(sums2_v, stage_sh.at[s])
    plsc.subcore_barrier()

    for s2 in range(NS):
        for i in range(HPC):
            pltpu.sync_copy(stage_sh.at[s2, i, pl.ds(s * NPS, NPS)],
                            sums_loc.at[s2, i])
    for i in range(HPC):
        gh = 2 * c + i
        ghv = jnp.full((L,), 0, _i32) + gh
        mg = plsc.load_gather(m_v, [ghv])
        escale = jnp.exp(-mg) * float(E)

        def _denom(g, _):
            sl = pl.ds(g * L, L)
            acc = escale
            for s2 in range(NS):
                acc = acc + sums_loc[s2, i, sl]
            dloc_v[i, sl] = acc
            return 0
        lax.fori_loop(0, NPS // L, _denom, 0)
        pltpu.sync_copy(dloc_v.at[i], denom_sh.at[i, pl.ds(s * NPS, NPS)])
    plsc.subcore_barrier()

    pltpu.sync_copy(denom_sh, denom_v)
    for i in range(HPC):
        iv = jnp.full((L,), i, _i32)

        def _att(g, _):
            sl = pl.ds(g * L, L)
            tv = tgt_v[sl]
            dv = plsc.load_gather(denom_v, [iv, tv])
            att_v[i, sl] = exp_v[i, sl] / dv
            return 0
        lax.fori_loop(0, EC // L, _att, 0)

    for d in descs:
        d.wait()

    def _scale(g, _):
        sl = pl.ds(g * L, L)
        a0v = att_v[0, sl]
        a1v = att_v[1, sl]
        avs = []
        for k in range(NKC):
            gh = k // HPC
            avs.append(jnp.where(gh == 2 * c, a0v,
                                 jnp.where(gh == 2 * c + 1, a1v, zero16)))
        for r in range(L):
            j = g * L + r
            for k in range(NKC):
                ksl = pl.ds(k * L, L)
                rows_v[j, ksl] = rows_v[j, ksl] * avs[k][r]
        return 0
    lax.fori_loop(0, EC // L, _scale, 0)

    for k in range(KCH):
        pltpu.sync_copy(rows_v.at[pl.ds(k * 128, 128)],
                        hp_sh.at[tgt_idx.at[k]], add=True)
    plsc.subcore_barrier()

    pltpu.sync_copy(hp_sh.at[pl.ds(s * NPS, NPS)],
                    out_hbm.at[c, pl.ds(s * NPS, NPS)])


@functools.lru_cache(maxsize=None)
def _build_sc_edges():
  return pl.kernel(
    _sc_body,
    out_type=jax.ShapeDtypeStruct((NC, N, ND), _f32),
    mesh=plsc.VectorSubcoreMesh(core_axis_name="c", subcore_axis_name="s",
                                num_cores=NC, num_subcores=NS),
    compiler_params=pltpu.CompilerParams(needs_layout_passes=False),
    scratch_types=[
        pltpu.VMEM((EC,), _i32),
        pltpu.VMEM((EC,), _i32),
        pltpu.VMEM((KCH, 128), _i32),
        pltpu.VMEM((KCH, 128), _i32),
        pltpu.VMEM((N * 2 * H,), _f32),
        pltpu.VMEM((EC * H,), _f32),
        pltpu.VMEM((L,), _f32),
        pltpu.VMEM((HPC, EC), _f32),
        pltpu.VMEM((HPC, EC), _f32),
        pltpu.VMEM((HPC, N), _f32),
        pltpu.VMEM((NS, HPC, NPS), _f32),
        pltpu.VMEM((HPC, NPS), _f32),
        pltpu.VMEM((HPC, N), _f32),
        pltpu.VMEM((EC, ND), _f32),
        pltpu.VMEM_SHARED((NS, HPC, N), _f32),
        pltpu.VMEM_SHARED((HPC, N), _f32),
        pltpu.VMEM_SHARED((N, ND), _f32),
        pltpu.SemaphoreType.DMA,
    ],
  )


@jax.jit
def kernel(node_features, edge_indices, edge_features, W_node, W_edge, a):
    h, alpha, ae, m = _prep(node_features, W_node, edge_features, W_edge, a)
    src = edge_indices[0]
    tgt = edge_indices[1]
    out = _build_sc_edges()(src, tgt, h, alpha.reshape(-1), ae.reshape(-1),
                            m.reshape(-1))
    return jnp.concatenate([out[0][:, :CW], out[1][:, CW:]], axis=1)


# speedup vs baseline: 21.9572x; 1.2472x over previous
import functools

import jax
import jax.numpy as jnp
from jax import lax
from jax.experimental import pallas as pl
from jax.experimental.pallas import tpu as pltpu
from jax.experimental.pallas import tpu_sc as plsc

N, E, ND, ED, H, HD = 2048, 8192, 128, 16, 4, 32
NC, NS, L = 2, 16, 16
EC = E // NS
NPS = N // NS
HPC = H // NC
CW = HPC * HD
KCH = EC // 128
GPC = 128 // L
NKC = ND // L

_f32 = jnp.float32
_i32 = jnp.int32


def _prep_body(nf_ref, wn_ref, ef_ref, we_ref, a_ref,
               h_ref, alpha_ref, ae_ref, m_ref):
    nf = nf_ref[...]
    wn = wn_ref[...]
    av = a_ref[...]
    h = jnp.dot(nf, wn, preferred_element_type=_f32)
    h_ref[...] = h

    wn3 = wn.reshape(ND, H, HD)
    a_s = av[:, 0:HD]
    a_t = av[:, HD:2 * HD]
    a_e = av[:, 2 * HD:3 * HD]
    As = (wn3 * a_s[None, :, :]).sum(axis=-1)
    At = (wn3 * a_t[None, :, :]).sum(axis=-1)
    Ast = jnp.concatenate([As, At], axis=1)
    alphaT = lax.dot_general(Ast, nf, (((0,), (1,)), ((), ())),
                             preferred_element_type=_f32)
    alpha_ref[...] = alphaT

    we3 = we_ref[...].reshape(ED, H, HD)
    Ae = (we3 * a_e[None, :, :]).sum(axis=-1)
    aeT = lax.dot_general(Ae, ef_ref[...], (((0,), (1,)), ((), ())),
                          preferred_element_type=_f32)
    ae_ref[...] = aeT

    amax = jnp.max(alphaT, axis=1, keepdims=True)
    aemax = jnp.max(aeT, axis=1, keepdims=True)
    M = jnp.maximum(amax[0:H] + amax[H:2 * H] + aemax, 0.0)
    M = jnp.minimum(M, 80.0).reshape(1, H)
    m_ref[...] = jnp.concatenate([M, jnp.zeros((1, ND - H), _f32)], axis=1)


_prep = pl.pallas_call(
    _prep_body,
    out_shape=[
        jax.ShapeDtypeStruct((N, ND), _f32),
        jax.ShapeDtypeStruct((2 * H, N), _f32),
        jax.ShapeDtypeStruct((H, E), _f32),
        jax.ShapeDtypeStruct((1, ND), _f32),
    ],
)


def _sc_body(ei_hbm, h_hbm, alpha_hbm, ae_hbm, m_hbm,
             out_hbm,
             src_v, tgt_v, src_idx, tgt_idx, alpha_v, ae_v, m_v,
             exp_v, att_v, sums2_v, sums_loc, dloc_v, denom_v, rows_v,
             stage_sh, denom_sh, hp_sh, sem):
    c = lax.axis_index("c")
    s = lax.axis_index("s")
    base = s * EC
    zero16 = jnp.zeros((L,), _f32)
    iota = lax.iota(_i32, L)

    pltpu.sync_copy(ei_hbm.at[0, pl.ds(base, EC)], src_v)
    pltpu.sync_copy(ei_hbm.at[1, pl.ds(base, EC)], tgt_v)
    pltpu.sync_copy(alpha_hbm, alpha_v)
    for i in range(HPC):
        pltpu.sync_copy(ae_hbm.at[2 * c + i, pl.ds(base, EC)], ae_v.at[i])
    pltpu.sync_copy(m_hbm, m_v)

    def _zero_sums(g, _):
        for i in range(HPC):
            sums2_v[i, pl.ds(g * L, L)] = zero16
        return 0
    lax.fori_loop(0, N // L, _zero_sums, 0)

    def _zero_rows(j, _):
        for k in range(NKC):
            rows_v[j, pl.ds(k * L, L)] = zero16
        return 0
    lax.fori_loop(0, NPS, _zero_rows, 0)
    pltpu.sync_copy(rows_v.at[pl.ds(0, NPS)], hp_sh.at[pl.ds(s * NPS, NPS)])

    def _fill_idx(g, _):
        k = g // GPC
        r = g - k * GPC
        sl = pl.ds(k * 128 + r * L, L)
        dsl = pl.ds(r * L, L)
        src_idx[k, dsl] = src_v[sl]
        tgt_idx[k, dsl] = tgt_v[sl]
        return 0
    lax.fori_loop(0, EC // L, _fill_idx, 0)

    descs = [pltpu.async_copy(h_hbm.at[src_idx.at[k]],
                              rows_v.at[pl.ds(k * 128, 128)], sem)
             for k in range(KCH)]

    plsc.subcore_barrier()

    zi = jnp.full((L,), 0, _i32)
    for i in range(HPC):
        gh = 2 * c + i
        ghv = zi + gh
        mg = plsc.load_gather(m_v, [zi, ghv])
        eneg = jnp.exp(-mg)
        iv = jnp.full((L,), i, _i32)

        def _logit(g, _):
            sl = pl.ds(g * L, L)
            sv = src_v[sl]
            tv = tgt_v[sl]
            jv = iota + g * L
            asv = plsc.load_gather(alpha_v, [ghv, sv])
            atv = plsc.load_gather(alpha_v, [ghv + H, tv])
            aev = ae_v[i, sl]
            l = asv + atv + aev
            l = jnp.where(l > 0.0, l, l * 0.2)
            ex = jnp.exp(l - mg)
            exp_v[i, sl] = ex
            plsc.addupdate_scatter(sums2_v, [iv, tv], ex - eneg)
            return 0
        lax.fori_loop(0, EC // L, _logit, 0)

    pltpu.sync_copy(sums2_v, stage_sh.at[s])
    plsc.subcore_barrier()

    for s2 in range(NS):
        for i in range(HPC):
            pltpu.sync_copy(stage_sh.at[s2, i, pl.ds(s * NPS, NPS)],
                            sums_loc.at[s2, i])
    for i in range(HPC):
        gh = 2 * c + i
        ghv = zi + gh
        mg = plsc.load_gather(m_v, [zi, ghv])
        escale = jnp.exp(-mg) * float(E)

        def _denom(g, _):
            sl = pl.ds(g * L, L)
            acc = escale
            for s2 in range(NS):
                acc = acc + sums_loc[s2, i, sl]
            dloc_v[i, sl] = acc
            return 0
        lax.fori_loop(0, NPS // L, _denom, 0)
        pltpu.sync_copy(dloc_v.at[i], denom_sh.at[i, pl.ds(s * NPS, NPS)])
    plsc.subcore_barrier()

    pltpu.sync_copy(denom_sh, denom_v)
    for i in range(HPC):
        iv = jnp.full((L,), i, _i32)

        def _att(g, _):
            sl = pl.ds(g * L, L)
            tv = tgt_v[sl]
            dv = plsc.load_gather(denom_v, [iv, tv])
            att_v[i, sl] = exp_v[i, sl] / dv
            return 0
        lax.fori_loop(0, EC // L, _att, 0)

    for d in descs:
        d.wait()

    def _scale(g, _):
        sl = pl.ds(g * L, L)
        a0v = att_v[0, sl]
        a1v = att_v[1, sl]
        avs = []
        for k in range(NKC):
            gh = k // HPC
            avs.append(jnp.where(gh == 2 * c, a0v,
                                 jnp.where(gh == 2 * c + 1, a1v, zero16)))
        for r in range(L):
            j = g * L + r
            for k in range(NKC):
                ksl = pl.ds(k * L, L)
                rows_v[j, ksl] = rows_v[j, ksl] * avs[k][r]
        return 0
    lax.fori_loop(0, EC // L, _scale, 0)

    for k in range(KCH):
        pltpu.sync_copy(rows_v.at[pl.ds(k * 128, 128)],
                        hp_sh.at[tgt_idx.at[k]], add=True)
    plsc.subcore_barrier()

    pltpu.sync_copy(hp_sh.at[pl.ds(s * NPS, NPS)],
                    out_hbm.at[c, pl.ds(s * NPS, NPS)])


@functools.lru_cache(maxsize=None)
def _build_sc_edges():
  return pl.kernel(
    _sc_body,
    out_type=jax.ShapeDtypeStruct((NC, N, ND), _f32),
    mesh=plsc.VectorSubcoreMesh(core_axis_name="c", subcore_axis_name="s",
                                num_cores=NC, num_subcores=NS),
    compiler_params=pltpu.CompilerParams(needs_layout_passes=False),
    scratch_types=[
        pltpu.VMEM((EC,), _i32),
        pltpu.VMEM((EC,), _i32),
        pltpu.VMEM((KCH, 128), _i32),
        pltpu.VMEM((KCH, 128), _i32),
        pltpu.VMEM((2 * H, N), _f32),
        pltpu.VMEM((HPC, EC), _f32),
        pltpu.VMEM((1, ND), _f32),
        pltpu.VMEM((HPC, EC), _f32),
        pltpu.VMEM((HPC, EC), _f32),
        pltpu.VMEM((HPC, N), _f32),
        pltpu.VMEM((NS, HPC, NPS), _f32),
        pltpu.VMEM((HPC, NPS), _f32),
        pltpu.VMEM((HPC, N), _f32),
        pltpu.VMEM((EC, ND), _f32),
        pltpu.VMEM_SHARED((NS, HPC, N), _f32),
        pltpu.VMEM_SHARED((HPC, N), _f32),
        pltpu.VMEM_SHARED((N, ND), _f32),
        pltpu.SemaphoreType.DMA,
    ],
  )


@jax.jit
def kernel(node_features, edge_indices, edge_features, W_node, W_edge, a):
    h, alpha, ae, m = _prep(node_features, W_node, edge_features, W_edge, a)
    out = _build_sc_edges()(edge_indices, h, alpha, ae, m)
    return out[0] + out[1]


# speedup vs baseline: 21.9904x; 1.0015x over previous
import functools

import jax
import jax.numpy as jnp
from jax import lax
from jax.experimental import pallas as pl
from jax.experimental.pallas import tpu as pltpu
from jax.experimental.pallas import tpu_sc as plsc

N, E, ND, ED, H, HD = 2048, 8192, 128, 16, 4, 32
NC, NS, L = 2, 16, 16
EC = E // NS
NPS = N // NS
HPC = H // NC
CW = HPC * HD
KCH = EC // 128
GPC = 128 // L
NKC = ND // L

_f32 = jnp.float32
_i32 = jnp.int32


def _prep_body(nf_ref, wn_ref, ef_ref, we_ref, a_ref,
               h_ref, alpha_ref, ae_ref, m_ref):
    nf = nf_ref[...]
    wn = wn_ref[...]
    av = a_ref[...]
    h = jnp.dot(nf, wn, preferred_element_type=_f32)
    h_ref[...] = h

    wn3 = wn.reshape(ND, H, HD)
    a_s = av[:, 0:HD]
    a_t = av[:, HD:2 * HD]
    a_e = av[:, 2 * HD:3 * HD]
    As = (wn3 * a_s[None, :, :]).sum(axis=-1)
    At = (wn3 * a_t[None, :, :]).sum(axis=-1)
    Ast = jnp.concatenate([As, At], axis=1)
    alphaT = lax.dot_general(Ast, nf, (((0,), (1,)), ((), ())),
                             preferred_element_type=_f32)
    alpha_ref[...] = alphaT

    we3 = we_ref[...].reshape(ED, H, HD)
    Ae = (we3 * a_e[None, :, :]).sum(axis=-1)
    aeT = lax.dot_general(Ae, ef_ref[...], (((0,), (1,)), ((), ())),
                          preferred_element_type=_f32)
    ae_ref[...] = aeT

    amax = jnp.max(alphaT, axis=1, keepdims=True)
    aemax = jnp.max(aeT, axis=1, keepdims=True)
    M = jnp.maximum(amax[0:H] + amax[H:2 * H] + aemax, 0.0)
    M = jnp.minimum(M, 80.0).reshape(1, H)
    m_ref[...] = jnp.concatenate([M, jnp.zeros((1, ND - H), _f32)], axis=1)


_prep = pl.pallas_call(
    _prep_body,
    out_shape=[
        jax.ShapeDtypeStruct((N, ND), _f32),
        jax.ShapeDtypeStruct((2 * H, N), _f32),
        jax.ShapeDtypeStruct((H, E), _f32),
        jax.ShapeDtypeStruct((1, ND), _f32),
    ],
)


def _sc_body(ei_hbm, h_hbm, alpha_hbm, ae_hbm, m_hbm,
             out_hbm,
             src_v, tgt_v, src_idx, tgt_idx, alpha_v, ae_v, m_v,
             exp_v, att_v, sums2_v, sums_loc, dloc_v, denom_v, rows_v,
             stage_sh, denom_sh, hp_sh, sem, sem_stage, sem_hp):
    c = lax.axis_index("c")
    s = lax.axis_index("s")
    base = s * EC
    zero16 = jnp.zeros((L,), _f32)
    iota = lax.iota(_i32, L)

    pltpu.sync_copy(ei_hbm.at[0, pl.ds(base, EC)], src_v)
    pltpu.sync_copy(ei_hbm.at[1, pl.ds(base, EC)], tgt_v)
    pltpu.sync_copy(alpha_hbm, alpha_v)
    for i in range(HPC):
        pltpu.sync_copy(ae_hbm.at[2 * c + i, pl.ds(base, EC)], ae_v.at[i])
    pltpu.sync_copy(m_hbm, m_v)

    def _zero_sums(g, _):
        for i in range(HPC):
            sums2_v[i, pl.ds(g * L, L)] = zero16
        return 0
    lax.fori_loop(0, N // L, _zero_sums, 0)

    def _zero_rows(j, _):
        for k in range(NKC):
            rows_v[j, pl.ds(k * L, L)] = zero16
        return 0
    lax.fori_loop(0, NPS, _zero_rows, 0)
    pltpu.sync_copy(rows_v.at[pl.ds(0, NPS)], hp_sh.at[pl.ds(s * NPS, NPS)])

    def _fill_idx(g, _):
        k = g // GPC
        r = g - k * GPC
        sl = pl.ds(k * 128 + r * L, L)
        dsl = pl.ds(r * L, L)
        src_idx[k, dsl] = src_v[sl]
        tgt_idx[k, dsl] = tgt_v[sl]
        return 0
    lax.fori_loop(0, EC // L, _fill_idx, 0)

    descs = [pltpu.async_copy(h_hbm.at[src_idx.at[k]],
                              rows_v.at[pl.ds(k * 128, 128)], sem)
             for k in range(KCH)]

    plsc.subcore_barrier()

    zi = jnp.full((L,), 0, _i32)
    for i in range(HPC):
        gh = 2 * c + i
        ghv = zi + gh
        mg = plsc.load_gather(m_v, [zi, ghv])
        eneg = jnp.exp(-mg)
        iv = jnp.full((L,), i, _i32)

        def _logit(g, _):
            sl = pl.ds(g * L, L)
            sv = src_v[sl]
            tv = tgt_v[sl]
            jv = iota + g * L
            asv = plsc.load_gather(alpha_v, [ghv, sv])
            atv = plsc.load_gather(alpha_v, [ghv + H, tv])
            aev = ae_v[i, sl]
            l = asv + atv + aev
            l = jnp.where(l > 0.0, l, l * 0.2)
            ex = jnp.exp(l - mg)
            exp_v[i, sl] = ex
            plsc.addupdate_scatter(sums2_v, [iv, tv], ex - eneg)
            return 0
        lax.fori_loop(0, EC // L, _logit, 0)

    pltpu.sync_copy(sums2_v, stage_sh.at[s])
    plsc.subcore_barrier()

    stage_descs = [pltpu.async_copy(stage_sh.at[s2, i, pl.ds(s * NPS, NPS)],
                                    sums_loc.at[s2, i], sem_stage)
                   for s2 in range(NS) for i in range(HPC)]
    for d in stage_descs:
        d.wait()
    for i in range(HPC):
        gh = 2 * c + i
        ghv = zi + gh
        mg = plsc.load_gather(m_v, [zi, ghv])
        escale = jnp.exp(-mg) * float(E)

        def _denom(g, _):
            sl = pl.ds(g * L, L)
            acc = escale
            for s2 in range(NS):
                acc = acc + sums_loc[s2, i, sl]
            dloc_v[i, sl] = acc
            return 0
        lax.fori_loop(0, NPS // L, _denom, 0)
        pltpu.sync_copy(dloc_v.at[i], denom_sh.at[i, pl.ds(s * NPS, NPS)])
    plsc.subcore_barrier()

    pltpu.sync_copy(denom_sh, denom_v)
    for i in range(HPC):
        iv = jnp.full((L,), i, _i32)

        def _att(g, _):
            sl = pl.ds(g * L, L)
            tv = tgt_v[sl]
            dv = plsc.load_gather(denom_v, [iv, tv])
            att_v[i, sl] = exp_v[i, sl] / dv
            return 0
        lax.fori_loop(0, EC // L, _att, 0)

    for d in descs:
        d.wait()

    def _scale(g, _):
        sl = pl.ds(g * L, L)
        a0v = att_v[0, sl]
        a1v = att_v[1, sl]
        avs = []
        for k in range(NKC):
            gh = k // HPC
            avs.append(jnp.where(gh == 2 * c, a0v,
                                 jnp.where(gh == 2 * c + 1, a1v, zero16)))
        for r in range(L):
            j = g * L + r
            for k in range(NKC):
                ksl = pl.ds(k * L, L)
                rows_v[j, ksl] = rows_v[j, ksl] * avs[k][r]
        return 0
    lax.fori_loop(0, EC // L, _scale, 0)

    hp_descs = [pltpu.async_copy(rows_v.at[pl.ds(k * 128, 128)],
                                 hp_sh.at[tgt_idx.at[k]], sem_hp, add=True)
                for k in range(KCH)]
    for d in hp_descs:
        d.wait()
    plsc.subcore_barrier()

    pltpu.sync_copy(hp_sh.at[pl.ds(s * NPS, NPS)],
                    out_hbm.at[c, pl.ds(s * NPS, NPS)])


@functools.lru_cache(maxsize=None)
def _build_sc_edges():
  return pl.kernel(
    _sc_body,
    out_type=jax.ShapeDtypeStruct((NC, N, ND), _f32),
    mesh=plsc.VectorSubcoreMesh(core_axis_name="c", subcore_axis_name="s",
                                num_cores=NC, num_subcores=NS),
    compiler_params=pltpu.CompilerParams(needs_layout_passes=False),
    scratch_types=[
        pltpu.VMEM((EC,), _i32),
        pltpu.VMEM((EC,), _i32),
        pltpu.VMEM((KCH, 128), _i32),
        pltpu.VMEM((KCH, 128), _i32),
        pltpu.VMEM((2 * H, N), _f32),
        pltpu.VMEM((HPC, EC), _f32),
        pltpu.VMEM((1, ND), _f32),
        pltpu.VMEM((HPC, EC), _f32),
        pltpu.VMEM((HPC, EC), _f32),
        pltpu.VMEM((HPC, N), _f32),
        pltpu.VMEM((NS, HPC, NPS), _f32),
        pltpu.VMEM((HPC, NPS), _f32),
        pltpu.VMEM((HPC, N), _f32),
        pltpu.VMEM((EC, ND), _f32),
        pltpu.VMEM_SHARED((NS, HPC, N), _f32),
        pltpu.VMEM_SHARED((HPC, N), _f32),
        pltpu.VMEM_SHARED((N, ND), _f32),
        pltpu.SemaphoreType.DMA,
        pltpu.SemaphoreType.DMA,
        pltpu.SemaphoreType.DMA,
    ],
  )


@jax.jit
def kernel(node_features, edge_indices, edge_features, W_node, W_edge, a):
    h, alpha, ae, m = _prep(node_features, W_node, edge_features, W_edge, a)
    out = _build_sc_edges()(edge_indices, h, alpha, ae, m)
    return out[0] + out[1]
